# baseline (speedup 1.0000x reference)
import functools

import jax
import jax.numpy as jnp
from jax import lax
from jax.experimental import pallas as pl
from jax.experimental.pallas import tpu as pltpu
from jax.experimental.pallas import tpu_sc as plsc

H = 64
SUB = 128
KS = (9, 70, 200, 50)
NB = 3
KMAX = max(KS)


@functools.lru_cache(maxsize=None)
def _make_embed_call(batch_size):
    info = plsc.get_sparse_core_info()
    nc, ns = info.num_cores, info.num_subcores
    nw = nc * ns
    assert batch_size == nw * SUB
    nwt = batch_size // SUB

    mesh = plsc.VectorSubcoreMesh(core_axis_name="c", subcore_axis_name="s")

    out_type = tuple(
        jax.ShapeDtypeStruct((k * 8 * nwt * 8 * SUB,), jnp.float32) for k in KS
    )

    @functools.partial(
        pl.kernel,
        mesh=mesh,
        out_type=out_type,
        scratch_types=[
            pltpu.VMEM((KMAX, SUB), jnp.int32),
            pltpu.VMEM((NB, SUB, H), jnp.float32),
            pltpu.VMEM((NB, 8 * 8 * SUB), jnp.float32),
            pltpu.SemaphoreType.DMA,
            pltpu.SemaphoreType.DMA((NB,)),
            pltpu.SemaphoreType.DMA((NB,)),
        ],
        compiler_params=pltpu.CompilerParams(use_tc_tiling_on_sc=False,
                                             needs_layout_passes=False),
    )
    def embed(demo_i, vital_i, dx_i, proc_i,
              demo_t, vital_t, dx_t, proc_t,
              demo_o, vital_o, dx_o, proc_o,
              idx_v, rows, tr, isem, gsem, ssem):
        wid = lax.axis_index("s") * nc + lax.axis_index("c")
        iota = lax.iota(jnp.int32, 16)
        hbase = [SUB * (16 * hc + iota) for hc in range(4)]

        for (idx_t_hbm, tbl, out_hbm, k) in (
            (demo_i, demo_t, demo_o, KS[0]),
            (vital_i, vital_t, vital_o, KS[1]),
            (dx_i, dx_t, dx_o, KS[2]),
            (proc_i, proc_t, proc_o, KS[3]),
        ):
            def fetch(t, carry, idx_t_hbm=idx_t_hbm):
                pltpu.async_copy(
                    idx_t_hbm.at[t, pl.ds(wid * SUB, SUB)], idx_v.at[t], isem)
                return carry

            lax.fori_loop(0, k, fetch, 0)
            pltpu.make_async_copy(
                idx_t_hbm.at[pl.ds(0, k), pl.ds(0, SUB)],
                idx_v.at[pl.ds(0, k)], isem).wait()

            ngrp = (k + NB - 1) // NB

            def grp(g, carry, tbl=tbl, out_hbm=out_hbm, k=k):
                for b in range(NB):
                    s = g * NB + b

                    @pl.when(jnp.logical_and(s < k, s >= NB))
                    def _(b=b, out_hbm=out_hbm):
                        pltpu.make_async_copy(
                            tr.at[b], out_hbm.at[pl.ds(0, 8 * 8 * SUB)],
                            ssem.at[b]).wait()

                    @pl.when(s < k)
                    def _(b=b, s=s, tbl=tbl):
                        pltpu.async_copy(
                            tbl.at[idx_v.at[s]], rows.at[b], gsem.at[b])
                for b in range(NB):
                    s = g * NB + b

                    @pl.when(s < k)
                    def _(b=b, s=s, tbl=tbl, out_hbm=out_hbm):
                        pltpu.make_async_copy(
                            tbl.at[pl.ds(0, SUB)], rows.at[b],
                            gsem.at[b]).wait()

                        @plsc.parallel_loop(0, SUB, unroll=16)
                        def _(l, b=b):
                            vl = jnp.full((16,), l, jnp.int32)
                            for hc in range(4):
                                x = rows.at[b][l, pl.ds(16 * hc, 16)]
                                plsc.store_scatter(
                                    tr.at[b], [hbase[hc] + vl], x)
                        tile0 = (8 * s * nwt + wid) * (8 * SUB)
                        for hb in range(8):
                            pltpu.async_copy(
                                tr.at[b, pl.ds(hb * 8 * SUB, 8 * SUB)],
                                out_hbm.at[pl.ds(
                                    tile0 + hb * nwt * 8 * SUB, 8 * SUB)],
                                ssem.at[b])
                return carry

            lax.fori_loop(0, ngrp, grp, 0)
            for b in range(NB):
                pltpu.make_async_copy(
                    tr.at[b], out_hbm.at[pl.ds(0, 8 * 8 * SUB)],
                    ssem.at[b]).wait()

    return embed


def kernel(demographics_ints, vital_signs_ints, dx_ints, proc_ints,
           demo_table, vital_table, dx_table, proc_table, visit_table):
    batch_size = demographics_ints.shape[0]
    embed = _make_embed_call(batch_size)
    idx_ts = [x.astype(jnp.int32).T
              for x in (demographics_ints, vital_signs_ints,
                        dx_ints, proc_ints)]
    tiles = embed(idx_ts[0], idx_ts[1], idx_ts[2], idx_ts[3],
                  demo_table, vital_table, dx_table, proc_table)
    nwt = batch_size // SUB
    outs = []
    for y, k in zip(tiles, KS):
        y4 = y.reshape(k * 8, nwt, 8, SUB)
        m = y4.transpose((0, 2, 1, 3)).reshape(k * H, batch_size)
        outs.append(m.T.reshape(batch_size, k, H))
    demo_emb, vital_emb, dx_emb, proc_emb = outs
    visit_emb = jnp.broadcast_to(visit_table[None, :, :],
                                 (batch_size, 1, visit_table.shape[1]))
    mask_visit = jnp.ones((batch_size, 1), dtype=jnp.float32)
    mask_demo = jnp.ones((batch_size, KS[0]), dtype=jnp.float32)
    mask_vital = jnp.ones((batch_size, KS[1]), dtype=jnp.float32)
    return (demo_emb, vital_emb, dx_emb, proc_emb, visit_emb,
            mask_visit, mask_demo, mask_vital)


# speedup vs baseline: 1.1949x; 1.1949x over previous
import functools

import jax
import jax.numpy as jnp
from jax import lax
from jax.experimental import pallas as pl
from jax.experimental.pallas import tpu as pltpu
from jax.experimental.pallas import tpu_sc as plsc

H = 64
SUB = 128
KS = (9, 70, 200, 50)
NB = 4
KMAX = max(KS)


@functools.lru_cache(maxsize=None)
def _make_embed_call(batch_size):
    info = plsc.get_sparse_core_info()
    nc, ns = info.num_cores, info.num_subcores
    nw = nc * ns
    assert batch_size == nw * SUB

    mesh = plsc.VectorSubcoreMesh(core_axis_name="c", subcore_axis_name="s")

    out_type = tuple(
        jax.ShapeDtypeStruct((batch_size * k, H), jnp.float32) for k in KS
    )

    @functools.partial(
        pl.kernel,
        mesh=mesh,
        out_type=out_type,
        scratch_types=[
            pltpu.VMEM((KMAX, SUB), jnp.int32),
            pltpu.VMEM((NB, SUB, H), jnp.float32),
            pltpu.VMEM((NB, SUB), jnp.int32),
            pltpu.VMEM((SUB,), jnp.int32),
            pltpu.SemaphoreType.DMA,
            pltpu.SemaphoreType.DMA((NB,)),
            pltpu.SemaphoreType.DMA((NB,)),
        ],
        compiler_params=pltpu.CompilerParams(use_tc_tiling_on_sc=False),
    )
    def embed(demo_i, vital_i, dx_i, proc_i,
              demo_t, vital_t, dx_t, proc_t,
              demo_o, vital_o, dx_o, proc_o,
              idx_v, rows, oidx, pk, isem, gsem, ssem):
        wid = lax.axis_index("s") * nc + lax.axis_index("c")
        iota = lax.iota(jnp.int32, 16)

        for (idx_t_hbm, tbl, out_hbm, k) in (
            (demo_i, demo_t, demo_o, KS[0]),
            (vital_i, vital_t, vital_o, KS[1]),
            (dx_i, dx_t, dx_o, KS[2]),
            (proc_i, proc_t, proc_o, KS[3]),
        ):
            def fetch(t, carry, idx_t_hbm=idx_t_hbm):
                pltpu.async_copy(
                    idx_t_hbm.at[t, pl.ds(wid * SUB, SUB)], idx_v.at[t], isem)
                return carry

            lax.fori_loop(0, k, fetch, 0)
            pltpu.make_async_copy(
                idx_t_hbm.at[pl.ds(0, k), pl.ds(0, SUB)],
                idx_v.at[pl.ds(0, k)], isem).wait()

            for j in range(8):
                pk[pl.ds(16 * j, 16)] = (wid * SUB + 16 * j + iota) * k

            ngrp = (k + NB - 1) // NB

            def grp(g, carry, tbl=tbl, out_hbm=out_hbm, k=k):
                for b in range(NB):
                    s = g * NB + b

                    @pl.when(jnp.logical_and(s < k, s >= NB))
                    def _(b=b, out_hbm=out_hbm):
                        pltpu.make_async_copy(
                            rows.at[b], out_hbm.at[pl.ds(0, SUB)],
                            ssem.at[b]).wait()

                    @pl.when(s < k)
                    def _(b=b, s=s, tbl=tbl):
                        for j in range(8):
                            oidx.at[b][pl.ds(16 * j, 16)] = (
                                pk[pl.ds(16 * j, 16)] + s)
                        pltpu.async_copy(
                            tbl.at[idx_v.at[s]], rows.at[b], gsem.at[b])
                for b in range(NB):
                    s = g * NB + b

                    @pl.when(s < k)
                    def _(b=b, s=s, tbl=tbl, out_hbm=out_hbm):
                        pltpu.make_async_copy(
                            tbl.at[pl.ds(0, SUB)], rows.at[b],
                            gsem.at[b]).wait()
                        pltpu.async_copy(
                            rows.at[b], out_hbm.at[oidx.at[b]], ssem.at[b])
                return carry

            lax.fori_loop(0, ngrp, grp, 0)
            for b in range(NB):
                pltpu.make_async_copy(
                    rows.at[b], out_hbm.at[pl.ds(0, SUB)], ssem.at[b]).wait()

    return embed


def kernel(demographics_ints, vital_signs_ints, dx_ints, proc_ints,
           demo_table, vital_table, dx_table, proc_table, visit_table):
    batch_size = demographics_ints.shape[0]
    embed = _make_embed_call(batch_size)
    idx_ts = [x.astype(jnp.int32).T
              for x in (demographics_ints, vital_signs_ints,
                        dx_ints, proc_ints)]
    flats = embed(idx_ts[0], idx_ts[1], idx_ts[2], idx_ts[3],
                  demo_table, vital_table, dx_table, proc_table)
    outs = [f.reshape(batch_size, k, H) for f, k in zip(flats, KS)]
    demo_emb, vital_emb, dx_emb, proc_emb = outs
    visit_emb = jnp.broadcast_to(visit_table[None, :, :],
                                 (batch_size, 1, visit_table.shape[1]))
    mask_visit = jnp.ones((batch_size, 1), dtype=jnp.float32)
    mask_demo = jnp.ones((batch_size, KS[0]), dtype=jnp.float32)
    mask_vital = jnp.ones((batch_size, KS[1]), dtype=jnp.float32)
    return (demo_emb, vital_emb, dx_emb, proc_emb, visit_emb,
            mask_visit, mask_demo, mask_vital)


# speedup vs baseline: 1.3200x; 1.1048x over previous
import functools

import jax
import jax.numpy as jnp
from jax import lax
from jax.experimental import pallas as pl
from jax.experimental.pallas import tpu as pltpu
from jax.experimental.pallas import tpu_sc as plsc

H = 64
SUB = 128
KS = (9, 70, 200, 50)
NB = 4
KMAX = max(KS)


@functools.lru_cache(maxsize=None)
def _make_embed_call(batch_size, k):
    info = plsc.get_sparse_core_info()
    nc, ns = info.num_cores, info.num_subcores
    nw = nc * ns
    assert batch_size == nw * SUB

    mesh = plsc.VectorSubcoreMesh(core_axis_name="c", subcore_axis_name="s")

    out_type = jax.ShapeDtypeStruct((batch_size * k, H), jnp.float32)

    @functools.partial(
        pl.kernel,
        mesh=mesh,
        out_type=out_type,
        scratch_types=[
            pltpu.VMEM((k, SUB), jnp.int32),
            pltpu.VMEM((NB, SUB, H), jnp.float32),
            pltpu.VMEM((NB, SUB), jnp.int32),
            pltpu.VMEM((SUB,), jnp.int32),
            pltpu.SemaphoreType.DMA,
            pltpu.SemaphoreType.DMA((NB,)),
            pltpu.SemaphoreType.DMA((NB,)),
        ],
        compiler_params=pltpu.CompilerParams(use_tc_tiling_on_sc=False),
    )
    def embed(idx_t_hbm, tbl, out_hbm,
              idx_v, rows, oidx, pk, isem, gsem, ssem):
        wid = lax.axis_index("s") * nc + lax.axis_index("c")
        iota = lax.iota(jnp.int32, 16)

        def fetch(t, carry):
            pltpu.async_copy(
                idx_t_hbm.at[t, pl.ds(wid * SUB, SUB)], idx_v.at[t], isem)
            return carry

        lax.fori_loop(0, k, fetch, 0)
        pltpu.make_async_copy(
            idx_t_hbm.at[pl.ds(0, k), pl.ds(0, SUB)],
            idx_v.at[pl.ds(0, k)], isem).wait()

        for j in range(8):
            pk[pl.ds(16 * j, 16)] = (wid * SUB + 16 * j + iota) * k

        ngrp = (k + NB - 1) // NB

        def grp(g, carry):
            for b in range(NB):
                s = g * NB + b

                @pl.when(jnp.logical_and(s < k, s >= NB))
                def _(b=b):
                    pltpu.make_async_copy(
                        rows.at[b], out_hbm.at[pl.ds(0, SUB)],
                        ssem.at[b]).wait()

                @pl.when(s < k)
                def _(b=b, s=s):
                    for j in range(8):
                        oidx.at[b][pl.ds(16 * j, 16)] = (
                            pk[pl.ds(16 * j, 16)] + s)
                    pltpu.async_copy(
                        tbl.at[idx_v.at[s]], rows.at[b], gsem.at[b])
            for b in range(NB):
                s = g * NB + b

                @pl.when(s < k)
                def _(b=b, s=s):
                    pltpu.make_async_copy(
                        tbl.at[pl.ds(0, SUB)], rows.at[b],
                        gsem.at[b]).wait()
                    pltpu.async_copy(
                        rows.at[b], out_hbm.at[oidx.at[b]], ssem.at[b])
            return carry

        lax.fori_loop(0, ngrp, grp, 0)
        for b in range(NB):
            pltpu.make_async_copy(
                rows.at[b], out_hbm.at[pl.ds(0, SUB)], ssem.at[b]).wait()

    return embed


def kernel(demographics_ints, vital_signs_ints, dx_ints, proc_ints,
           demo_table, vital_table, dx_table, proc_table, visit_table):
    batch_size = demographics_ints.shape[0]
    outs = []
    for ints, tbl, k in zip(
            (demographics_ints, vital_signs_ints, dx_ints, proc_ints),
            (demo_table, vital_table, dx_table, proc_table), KS):
        embed = _make_embed_call(batch_size, k)
        flat = embed(ints.astype(jnp.int32).T, tbl)
        outs.append(flat.reshape(batch_size, k, H))
    demo_emb, vital_emb, dx_emb, proc_emb = outs
    visit_emb = jnp.broadcast_to(visit_table[None, :, :],
                                 (batch_size, 1, visit_table.shape[1]))
    mask_visit = jnp.ones((batch_size, 1), dtype=jnp.float32)
    mask_demo = jnp.ones((batch_size, KS[0]), dtype=jnp.float32)
    mask_vital = jnp.ones((batch_size, KS[1]), dtype=jnp.float32)
    return (demo_emb, vital_emb, dx_emb, proc_emb, visit_emb,
            mask_visit, mask_demo, mask_vital)


# speedup vs baseline: 1.3203x; 1.0002x over previous
import functools

import jax
import jax.numpy as jnp
from jax import lax
from jax.experimental import pallas as pl
from jax.experimental.pallas import tpu as pltpu
from jax.experimental.pallas import tpu_sc as plsc

H = 64
SUB = 128
KS = (9, 70, 200, 50)
NB = 4
KMAX = max(KS)


@functools.lru_cache(maxsize=None)
def _make_embed_call(batch_size, k):
    info = plsc.get_sparse_core_info()
    nc, ns = info.num_cores, info.num_subcores
    nw = nc * ns
    assert batch_size == nw * SUB

    mesh = plsc.VectorSubcoreMesh(core_axis_name="c", subcore_axis_name="s")

    out_type = jax.ShapeDtypeStruct((batch_size * k, H), jnp.float32)

    @functools.partial(
        pl.kernel,
        mesh=mesh,
        out_type=out_type,
        scratch_types=[
            pltpu.VMEM((k, SUB), jnp.int32),
            pltpu.VMEM((NB, SUB, H), jnp.float32),
            pltpu.VMEM((NB, SUB), jnp.int32),
            pltpu.VMEM((SUB,), jnp.int32),
            pltpu.SemaphoreType.DMA,
            pltpu.SemaphoreType.DMA((NB,)),
            pltpu.SemaphoreType.DMA((NB,)),
        ],
        compiler_params=pltpu.CompilerParams(use_tc_tiling_on_sc=False),
    )
    def embed(idx_t_hbm, tbl, out_hbm,
              idx_v, rows, oidx, pk, isem, gsem, ssem):
        wid = lax.axis_index("s") * nc + lax.axis_index("c")
        iota = lax.iota(jnp.int32, 16)

        def fetch(t, carry):
            pltpu.async_copy(
                idx_t_hbm.at[t, pl.ds(wid * SUB, SUB)], idx_v.at[t], isem)
            return carry

        lax.fori_loop(0, k, fetch, 0)
        pltpu.make_async_copy(
            idx_t_hbm.at[pl.ds(0, k), pl.ds(0, SUB)],
            idx_v.at[pl.ds(0, k)], isem).wait()

        for j in range(8):
            pk[pl.ds(16 * j, 16)] = (wid * SUB + 16 * j + iota) * k

        ngrp = (k + NB - 1) // NB

        def grp(g, carry):
            for b in range(NB):
                s = g * NB + b

                @pl.when(jnp.logical_and(s < k, s >= NB))
                def _(b=b):
                    pltpu.make_async_copy(
                        rows.at[b], out_hbm.at[pl.ds(0, SUB)],
                        ssem.at[b]).wait()

                @pl.when(s < k)
                def _(b=b, s=s):
                    for j in range(8):
                        oidx.at[b][pl.ds(16 * j, 16)] = (
                            pk[pl.ds(16 * j, 16)] + s)
                    pltpu.async_copy(
                        tbl.at[idx_v.at[s]], rows.at[b], gsem.at[b])
            for b in range(NB):
                s = g * NB + b

                @pl.when(s < k)
                def _(b=b, s=s):
                    pltpu.make_async_copy(
                        tbl.at[pl.ds(0, SUB)], rows.at[b],
                        gsem.at[b]).wait()
                    pltpu.async_copy(
                        rows.at[b], out_hbm.at[oidx.at[b]], ssem.at[b])
            return carry

        lax.fori_loop(0, ngrp, grp, 0)
        for b in range(NB):
            pltpu.make_async_copy(
                rows.at[b], out_hbm.at[pl.ds(0, SUB)], ssem.at[b]).wait()

    return embed


def kernel(demographics_ints, vital_signs_ints, dx_ints, proc_ints,
           demo_table, vital_table, dx_table, proc_table, visit_table):
    batch_size = demographics_ints.shape[0]
    feats = {}
    for name, ints, tbl, k in (
            ("dx", dx_ints, dx_table, KS[2]),
            ("proc", proc_ints, proc_table, KS[3]),
            ("vital", vital_signs_ints, vital_table, KS[1]),
            ("demo", demographics_ints, demo_table, KS[0])):
        embed = _make_embed_call(batch_size, k)
        flat = embed(ints.astype(jnp.int32).T, tbl)
        feats[name] = flat.reshape(batch_size, k, H)
    demo_emb, vital_emb, dx_emb, proc_emb = (
        feats["demo"], feats["vital"], feats["dx"], feats["proc"])
    visit_emb = jnp.broadcast_to(visit_table[None, :, :],
                                 (batch_size, 1, visit_table.shape[1]))
    mask_visit = jnp.ones((batch_size, 1), dtype=jnp.float32)
    mask_demo = jnp.ones((batch_size, KS[0]), dtype=jnp.float32)
    mask_vital = jnp.ones((batch_size, KS[1]), dtype=jnp.float32)
    return (demo_emb, vital_emb, dx_emb, proc_emb, visit_emb,
            mask_visit, mask_demo, mask_vital)
